# Initial kernel scaffold; baseline (speedup 1.0000x reference)
import jax
import jax.numpy as jnp
from jax.experimental import pallas as pl


def kernel(x, emb_table, W, b):
    raise NotImplementedError("write your pallas kernel here")


# speedup vs baseline: 2.3802x; 2.3802x over previous
import functools

import jax
import jax.numpy as jnp
from jax import lax
from jax.experimental import pallas as pl
from jax.experimental.pallas import tpu as pltpu
from jax.experimental.pallas import tpu_sc as plsc

B = 16384
L = 50
D = 64
OUT = 64

NC = 2
NS = 16
NW = NC * NS
BAGS_PER_W = B // NW
CH = 16
NCHUNK = BAGS_PER_W // CH
GI = 100
NG = CH * L // GI
NVEC = D // 16


def _sc_pool_kernel(x2d_hbm, table_hbm, out_hbm, idx_v, rows_v, pooled_v, sem):
    wid = lax.axis_index("s") * NC + lax.axis_index("c")
    bag0 = wid * BAGS_PER_W

    def chunk_body(c, carry):
        bag_base = bag0 + c * CH
        irow = pl.multiple_of(bag_base // 2, NG)
        pltpu.sync_copy(x2d_hbm.at[pl.ds(irow, NG)], idx_v)
        copies = []
        for g in range(NG):
            copies.append(
                pltpu.async_copy(
                    table_hbm.at[idx_v.at[g]],
                    rows_v.at[pl.ds(g * GI, GI)],
                    sem,
                )
            )
        for cp in copies:
            cp.wait()

        def bag_body(j, carry2):
            r0 = j * L

            def l_body(l, acc):
                r = r0 + l
                return tuple(
                    acc[k] + rows_v[r, pl.ds(16 * k, 16)] for k in range(NVEC)
                )

            acc = lax.fori_loop(
                0, L, l_body,
                tuple(jnp.zeros((16,), jnp.float32) for _ in range(NVEC)),
            )
            slot = c * CH + j
            for k in range(NVEC):
                pooled_v[slot, pl.ds(16 * k, 16)] = acc[k] * (1.0 / L)
            return carry2

        return lax.fori_loop(0, CH, bag_body, carry)

    lax.fori_loop(0, NCHUNK, chunk_body, 0)
    pltpu.sync_copy(pooled_v, out_hbm.at[pl.ds(bag0, BAGS_PER_W)])


_sc_pool = functools.partial(
    pl.kernel,
    mesh=plsc.VectorSubcoreMesh(core_axis_name="c", subcore_axis_name="s"),
    out_type=jax.ShapeDtypeStruct((B, D), jnp.float32),
    scratch_types=[
        pltpu.VMEM((NG, GI), jnp.int32),
        pltpu.VMEM((CH * L, D), jnp.float32),
        pltpu.VMEM((BAGS_PER_W, D), jnp.float32),
        pltpu.SemaphoreType.DMA,
    ],
    compiler_params=pltpu.CompilerParams(use_tc_tiling_on_sc=False),
)(_sc_pool_kernel)


BLK = 512


def _tc_head_kernel(p_ref, wt_ref, b_ref, o_ref):
    y = jnp.dot(p_ref[...], wt_ref[...], preferred_element_type=jnp.float32)
    y = y + b_ref[...]
    y = y - jnp.max(y, axis=1, keepdims=True)
    e = jnp.exp(y)
    o_ref[...] = e / jnp.sum(e, axis=1, keepdims=True)


def _tc_head(pooled, wt, b2):
    return pl.pallas_call(
        _tc_head_kernel,
        grid=(B // BLK,),
        in_specs=[
            pl.BlockSpec((BLK, D), lambda i: (i, 0)),
            pl.BlockSpec((D, OUT), lambda i: (0, 0)),
            pl.BlockSpec((1, OUT), lambda i: (0, 0)),
        ],
        out_specs=pl.BlockSpec((BLK, OUT), lambda i: (i, 0)),
        out_shape=jax.ShapeDtypeStruct((B, OUT), jnp.float32),
    )(pooled, wt, b2)


def kernel(x, emb_table, W, b):
    x2d = x.astype(jnp.int32).reshape(B * L // GI, GI)
    pooled = _sc_pool(x2d, emb_table)
    return _tc_head(pooled, W.T, b.reshape(1, OUT))


# speedup vs baseline: 2.6398x; 1.1091x over previous
import functools

import jax
import jax.numpy as jnp
from jax import lax
from jax.experimental import pallas as pl
from jax.experimental.pallas import tpu as pltpu
from jax.experimental.pallas import tpu_sc as plsc

B = 16384
L = 50
D = 64
OUT = 64

NC = 2
NS = 16
NW = NC * NS
BAGS_PER_W = B // NW
CH = 8
NCHUNK = BAGS_PER_W // CH
NPAIR = NCHUNK // 2
NVEC = D // 16


def _sc_pool_kernel(x_hbm, table_hbm, out_hbm, idx_v, rows_v, pooled_v,
                    sem0, sem1):
    wid = lax.axis_index("s") * NC + lax.axis_index("c")
    bag0 = pl.multiple_of(wid * BAGS_PER_W, BAGS_PER_W)

    pltpu.sync_copy(x_hbm.at[pl.ds(bag0, BAGS_PER_W)], idx_v)

    def gather(c, buf, sem):
        for j in range(CH):
            pltpu.async_copy(
                table_hbm.at[idx_v.at[c * CH + j]],
                rows_v.at[buf, j],
                sem,
            )

    def pool(c, buf):
        def bag_body(j, carry):
            def l_body(l, acc):
                return tuple(
                    acc[k] + rows_v[buf, j, l, pl.ds(16 * k, 16)]
                    for k in range(NVEC)
                )

            acc = lax.fori_loop(
                0, L, l_body,
                tuple(jnp.zeros((16,), jnp.float32) for _ in range(NVEC)),
            )
            for k in range(NVEC):
                pooled_v[c * CH + j, pl.ds(16 * k, 16)] = acc[k] * (1.0 / L)
            return carry

        lax.fori_loop(0, CH, bag_body, 0)

    def drain(c, buf, sem):
        for j in range(CH):
            pltpu.make_async_copy(
                table_hbm.at[idx_v.at[c * CH + j]],
                rows_v.at[buf, j],
                sem,
            ).wait()

    gather(0, 0, sem0)

    def pair_body(t, carry):
        c0 = t * 2
        c1 = c0 + 1
        gather(c1, 1, sem1)
        drain(c0, 0, sem0)
        pool(c0, 0)
        @pl.when(t < NPAIR - 1)
        def _():
            gather(c0 + 2, 0, sem0)

        drain(c1, 1, sem1)
        pool(c1, 1)
        return carry

    lax.fori_loop(0, NPAIR, pair_body, 0)
    pltpu.sync_copy(pooled_v, out_hbm.at[pl.ds(bag0, BAGS_PER_W)])


_sc_pool = functools.partial(
    pl.kernel,
    mesh=plsc.VectorSubcoreMesh(core_axis_name="c", subcore_axis_name="s"),
    out_type=jax.ShapeDtypeStruct((B, D), jnp.float32),
    scratch_types=[
        pltpu.VMEM((BAGS_PER_W, L), jnp.int32),
        pltpu.VMEM((2, CH, L, D), jnp.float32),
        pltpu.VMEM((BAGS_PER_W, D), jnp.float32),
        pltpu.SemaphoreType.DMA,
        pltpu.SemaphoreType.DMA,
    ],
    compiler_params=pltpu.CompilerParams(use_tc_tiling_on_sc=False),
)(_sc_pool_kernel)


BLK = 512


def _tc_head_kernel(p_ref, wt_ref, b_ref, o_ref):
    y = jnp.dot(p_ref[...], wt_ref[...], preferred_element_type=jnp.float32)
    y = y + b_ref[...]
    y = y - jnp.max(y, axis=1, keepdims=True)
    e = jnp.exp(y)
    o_ref[...] = e / jnp.sum(e, axis=1, keepdims=True)


def _tc_head(pooled, wt, b2):
    return pl.pallas_call(
        _tc_head_kernel,
        grid=(B // BLK,),
        in_specs=[
            pl.BlockSpec((BLK, D), lambda i: (i, 0)),
            pl.BlockSpec((D, OUT), lambda i: (0, 0)),
            pl.BlockSpec((1, OUT), lambda i: (0, 0)),
        ],
        out_specs=pl.BlockSpec((BLK, OUT), lambda i: (i, 0)),
        out_shape=jax.ShapeDtypeStruct((B, OUT), jnp.float32),
    )(pooled, wt, b2)


def kernel(x, emb_table, W, b):
    pooled = _sc_pool(x.astype(jnp.int32), emb_table)
    return _tc_head(pooled, W.T, b.reshape(1, OUT))
